# Initial kernel scaffold; baseline (speedup 1.0000x reference)
import jax
import jax.numpy as jnp
from jax.experimental import pallas as pl


def kernel(x_h, x_g, edge_index, edge_attr, u, batch_h, W1a, b1a, W1b, b1b, W2a, b2a, W2b, b2b):
    raise NotImplementedError("write your pallas kernel here")


# speedup vs baseline: 1.2653x; 1.2653x over previous
import functools

import jax
import jax.numpy as jnp
from jax import lax
from jax.experimental import pallas as pl
from jax.experimental.pallas import tpu as pltpu

N_H = 128
N_G = 128
N_X = 16
N_U = 16
NB_EDGE = 2048
NB_NODE = 512


def _leaky(x):
    return jnp.where(x >= 0, x, 0.1 * x)


def _mlp1_body(g_ref, e_ref, w1a_ref, b1a_ref, w1b_ref, b1b_ref,
               m1_ref, m2_ref, m3_ref, m4_ref):
    x = jnp.concatenate([g_ref[...], e_ref[...]], axis=1)
    h = _leaky(jnp.dot(x, w1a_ref[...], preferred_element_type=jnp.float32)
               + b1a_ref[...])
    out = (jnp.dot(h, w1b_ref[...], preferred_element_type=jnp.float32)
           + b1b_ref[...])
    o2 = out * out
    m1_ref[...] = out
    m2_ref[...] = o2
    m3_ref[...] = o2 * out
    m4_ref[...] = o2 * o2


def _run_mlp1(G, edge_attr, W1a, b1a, W1b, b1b):
    E = G.shape[0]
    D1 = W1a.shape[0]
    grid = (pl.cdiv(E, NB_EDGE),)
    out_sds = jax.ShapeDtypeStruct((E, D1), jnp.float32)
    blk_e = pl.BlockSpec((NB_EDGE, D1), lambda i: (i, 0))
    full = lambda s: pl.BlockSpec(s, lambda i: tuple(0 for _ in s))
    return pl.pallas_call(
        _mlp1_body,
        grid=grid,
        in_specs=[
            pl.BlockSpec((NB_EDGE, N_G), lambda i: (i, 0)),
            pl.BlockSpec((NB_EDGE, N_X), lambda i: (i, 0)),
            full(W1a.shape),
            full((1, D1)),
            full(W1b.shape),
            full((1, D1)),
        ],
        out_specs=[blk_e, blk_e, blk_e, blk_e],
        out_shape=[out_sds] * 4,
    )(G, edge_attr, W1a, b1a.reshape(1, D1), W1b, b1b.reshape(1, D1))


def _fin_body(xh_ref, s1_ref, s2_ref, s3_ref, s4_ref, cnt_ref, bh_ref,
              u_ref, w2h_ref, w2n_ref, w2a_ref, w2b_ref, w2c_ref, w2d_ref,
              w2u_ref, b2a_ref, w2bb_ref, b2b_ref, out_ref):
    n = cnt_ref[...][:, 0:1]
    inv = 1.0 / jnp.maximum(n, 1.0)
    a = s1_ref[...] * inv
    m2 = s2_ref[...] * inv
    m3 = s3_ref[...] * inv
    m4 = s4_ref[...] * inv
    a2 = a * a
    v = 1e-6 + jnp.maximum(m2 - a2, 0.0)
    b = jnp.sqrt(v)
    c = (m3 - 3.0 * a * m2 + 2.0 * a * a2) / (v * b)
    d = (m4 - 4.0 * a * m3 + 6.0 * a2 * m2 - 3.0 * a2 * a2) / (v * v)
    bh = bh_ref[...]
    oh = (bh == lax.broadcasted_iota(jnp.int32, (bh.shape[0], 8), 1)
          ).astype(jnp.float32)
    ub = jnp.dot(oh, u_ref[...], preferred_element_type=jnp.float32)
    dot = lambda x, w: jnp.dot(x, w, preferred_element_type=jnp.float32)
    h = (dot(xh_ref[...], w2h_ref[...]) + n * w2n_ref[...]
         + dot(a, w2a_ref[...]) + dot(b, w2b_ref[...])
         + dot(c, w2c_ref[...]) + dot(d, w2d_ref[...])
         + dot(ub, w2u_ref[...]) + b2a_ref[...])
    h = _leaky(h)
    out_ref[...] = dot(h, w2bb_ref[...]) + b2b_ref[...]


def _run_finalize(x_h, S1, S2, S3, S4, cnt, batch_h, u,
                  W2a, b2a, W2b, b2b):
    N = x_h.shape[0]
    D1 = S1.shape[1]
    grid = (pl.cdiv(N, NB_NODE),)
    blk = lambda w: pl.BlockSpec((NB_NODE, w), lambda i: (i, 0))
    full = lambda s: pl.BlockSpec(s, lambda i: tuple(0 for _ in s))
    o = 0
    W2h = W2a[o:o + N_H]; o += N_H
    W2n = W2a[o:o + 1]; o += 1
    W2aa = W2a[o:o + D1]; o += D1
    W2ab = W2a[o:o + D1]; o += D1
    W2ac = W2a[o:o + D1]; o += D1
    W2ad = W2a[o:o + D1]; o += D1
    W2u = W2a[o:o + N_U]
    return pl.pallas_call(
        _fin_body,
        grid=grid,
        in_specs=[
            blk(N_H), blk(D1), blk(D1), blk(D1), blk(D1), blk(16),
            pl.BlockSpec((NB_NODE, 1), lambda i: (i, 0)),
            full(u.shape), full(W2h.shape), full(W2n.shape),
            full(W2aa.shape), full(W2ab.shape), full(W2ac.shape),
            full(W2ad.shape), full(W2u.shape), full((1, N_H)),
            full(W2b.shape), full((1, N_H)),
        ],
        out_specs=blk(N_H),
        out_shape=jax.ShapeDtypeStruct((N, N_H), jnp.float32),
    )(x_h, S1, S2, S3, S4, cnt, batch_h.reshape(N, 1), u,
      W2h, W2n, W2aa, W2ab, W2ac, W2ad, W2u, b2a.reshape(1, N_H),
      W2b, b2b.reshape(1, N_H))


def kernel(x_h, x_g, edge_index, edge_attr, u, batch_h,
           W1a, b1a, W1b, b1b, W2a, b2a, W2b, b2b):
    src = edge_index[0]
    tgt = edge_index[1]
    Nn = x_h.shape[0]

    G = x_g[tgt]
    m1, m2, m3, m4 = _run_mlp1(G, edge_attr, W1a, b1a, W1b, b1b)

    S1 = jax.ops.segment_sum(m1, src, num_segments=Nn)
    S2 = jax.ops.segment_sum(m2, src, num_segments=Nn)
    S3 = jax.ops.segment_sum(m3, src, num_segments=Nn)
    S4 = jax.ops.segment_sum(m4, src, num_segments=Nn)
    ones = jnp.ones((src.shape[0], 16), dtype=jnp.float32)
    cnt = jax.ops.segment_sum(ones, src, num_segments=Nn)

    return _run_finalize(x_h, S1, S2, S3, S4, cnt, batch_h, u,
                         W2a, b2a, W2b, b2b)


# speedup vs baseline: 3.4020x; 2.6888x over previous
import functools

import jax
import jax.numpy as jnp
from jax import lax
from jax.experimental import pallas as pl
from jax.experimental.pallas import tpu as pltpu
from jax.experimental.pallas import tpu_sc as plsc

N_H = 128
N_G = 128
N_X = 16
N_U = 16
NB_EDGE = 2048
NB_NODE = 512


def _leaky(x):
    return jnp.where(x >= 0, x, 0.1 * x)


def _mlp1_body(g_ref, e_ref, w1a_ref, b1a_ref, w1b_ref, b1b_ref,
               m1_ref, m2_ref, m3_ref, m4_ref, tl_ref):
    x = jnp.concatenate([g_ref[...], e_ref[...]], axis=1)
    h = _leaky(jnp.dot(x, w1a_ref[...], preferred_element_type=jnp.float32)
               + b1a_ref[...])
    out = (jnp.dot(h, w1b_ref[...], preferred_element_type=jnp.float32)
           + b1b_ref[...])
    nb = out.shape[0]
    o2 = out * out
    o3 = o2 * out
    o4 = o2 * o2
    m1_ref[...] = out[:, :128]
    m2_ref[...] = o2[:, :128]
    m3_ref[...] = o3[:, :128]
    m4_ref[...] = o4[:, :128]
    tl_ref[...] = jnp.concatenate(
        [out[:, 128:], o2[:, 128:], o3[:, 128:], o4[:, 128:],
         jnp.ones((nb, 1), jnp.float32),
         jnp.zeros((nb, 63), jnp.float32)], axis=1)


def _run_mlp1(G, edge_attr, W1a, b1a, W1b, b1b):
    E = G.shape[0]
    D1 = W1a.shape[0]
    grid = (pl.cdiv(E, NB_EDGE),)
    out_sds = jax.ShapeDtypeStruct((E, 128), jnp.float32)
    blk_e = pl.BlockSpec((NB_EDGE, 128), lambda i: (i, 0))
    full = lambda s: pl.BlockSpec(s, lambda i: tuple(0 for _ in s))
    return pl.pallas_call(
        _mlp1_body,
        grid=grid,
        in_specs=[
            pl.BlockSpec((NB_EDGE, N_G), lambda i: (i, 0)),
            pl.BlockSpec((NB_EDGE, N_X), lambda i: (i, 0)),
            full(W1a.shape),
            full((1, D1)),
            full(W1b.shape),
            full((1, D1)),
        ],
        out_specs=[blk_e] * 5,
        out_shape=[out_sds] * 5,
    )(G, edge_attr, W1a, b1a.reshape(1, D1), W1b, b1b.reshape(1, D1))


def _fin_body(xh_ref, s1_ref, s2_ref, s3_ref, s4_ref, t0_ref, t1_ref,
              bh_ref,
              u_ref, w2h_ref, w2n_ref, w2a_ref, w2b_ref, w2c_ref, w2d_ref,
              w2u_ref, b2a_ref, w2bb_ref, b2b_ref, out_ref):
    t = t0_ref[...] + t1_ref[...]
    n = t[:, 64:65]
    inv = 1.0 / jnp.maximum(n, 1.0)
    cat = lambda s, lo: jnp.concatenate([s, t[:, lo:lo + 16]], axis=1)
    a = cat(s1_ref[...], 0) * inv
    m2 = cat(s2_ref[...], 16) * inv
    m3 = cat(s3_ref[...], 32) * inv
    m4 = cat(s4_ref[...], 48) * inv
    a2 = a * a
    v = 1e-6 + jnp.maximum(m2 - a2, 0.0)
    b = jnp.sqrt(v)
    c = (m3 - 3.0 * a * m2 + 2.0 * a * a2) / (v * b)
    d = (m4 - 4.0 * a * m3 + 6.0 * a2 * m2 - 3.0 * a2 * a2) / (v * v)
    bh = bh_ref[...]
    oh = (bh == lax.broadcasted_iota(jnp.int32, (bh.shape[0], 8), 1)
          ).astype(jnp.float32)
    ub = jnp.dot(oh, u_ref[...], preferred_element_type=jnp.float32)
    dot = lambda x, w: jnp.dot(x, w, preferred_element_type=jnp.float32)
    h = (dot(xh_ref[...], w2h_ref[...]) + n * w2n_ref[...]
         + dot(a, w2a_ref[...]) + dot(b, w2b_ref[...])
         + dot(c, w2c_ref[...]) + dot(d, w2d_ref[...])
         + dot(ub, w2u_ref[...]) + b2a_ref[...])
    h = _leaky(h)
    out_ref[...] = dot(h, w2bb_ref[...]) + b2b_ref[...]


def _run_finalize(x_h, S1, S2, S3, S4, T0, T1, batch_h, u,
                  W2a, b2a, W2b, b2b, D1):
    N = x_h.shape[0]
    grid = (pl.cdiv(N, NB_NODE),)
    blk = lambda w: pl.BlockSpec((NB_NODE, w), lambda i: (i, 0))
    full = lambda s: pl.BlockSpec(s, lambda i: tuple(0 for _ in s))
    o = 0
    W2h = W2a[o:o + N_H]; o += N_H
    W2n = W2a[o:o + 1]; o += 1
    W2aa = W2a[o:o + D1]; o += D1
    W2ab = W2a[o:o + D1]; o += D1
    W2ac = W2a[o:o + D1]; o += D1
    W2ad = W2a[o:o + D1]; o += D1
    W2u = W2a[o:o + N_U]
    return pl.pallas_call(
        _fin_body,
        grid=grid,
        in_specs=[
            blk(N_H), blk(128), blk(128), blk(128), blk(128), blk(128),
            blk(128),
            pl.BlockSpec((NB_NODE, 1), lambda i: (i, 0)),
            full(u.shape), full(W2h.shape), full(W2n.shape),
            full(W2aa.shape), full(W2ab.shape), full(W2ac.shape),
            full(W2ad.shape), full(W2u.shape), full((1, N_H)),
            full(W2b.shape), full((1, N_H)),
        ],
        out_specs=blk(N_H),
        out_shape=jax.ShapeDtypeStruct((N, N_H), jnp.float32),
    )(x_h, S1, S2, S3, S4, T0, T1, batch_h.reshape(N, 1), u,
      W2h, W2n, W2aa, W2ab, W2ac, W2ad, W2u, b2a.reshape(1, N_H),
      W2b, b2b.reshape(1, N_H))


_SCAT_K = 256


def _scatter_body(src_ref, m1_ref, m2_ref, m3_ref, m4_ref, tl_ref,
                  z128_ref,
                  s1_ref, s2_ref, s3_ref, s4_ref, t0_ref, t1_ref,
                  acc, idx_v, rows_v):
    E = m1_ref.shape[0]
    Np = s1_ref.shape[0]
    n_tiles = 16
    n_chunks = E // _SCAT_K
    per_tile = (n_chunks + n_tiles - 1) // n_tiles
    rows_per_tile = Np // n_tiles
    cid = lax.axis_index("c")
    sid = lax.axis_index("s")
    node_sl = pl.ds(sid * rows_per_tile, rows_per_tile)

    def do_pass(m_ref, out_ref, stride, par, nch):
        pltpu.sync_copy(z128_ref, acc.at[node_sl, :])
        plsc.subcore_barrier()

        def chunk(i, carry):
            j = i * stride + par
            g = j * n_tiles + sid

            @pl.when(g < n_chunks)
            def _():
                esl = pl.ds(g * _SCAT_K, _SCAT_K)
                pltpu.sync_copy(src_ref.at[esl], idx_v)
                pltpu.sync_copy(m_ref.at[esl, :], rows_v)
                pltpu.sync_copy(rows_v, acc.at[idx_v], add=True)

            return carry

        lax.fori_loop(0, nch, chunk, 0)
        plsc.subcore_barrier()
        pltpu.sync_copy(acc.at[node_sl, :], out_ref.at[node_sl, :])

    @pl.when(cid == 0)
    def _():
        do_pass(m1_ref, s1_ref, 1, 0, per_tile)
        do_pass(m2_ref, s2_ref, 1, 0, per_tile)
        do_pass(tl_ref, t0_ref, 2, 0, (per_tile + 1) // 2)

    @pl.when(cid == 1)
    def _():
        do_pass(m3_ref, s3_ref, 1, 0, per_tile)
        do_pass(m4_ref, s4_ref, 1, 0, per_tile)
        do_pass(tl_ref, t1_ref, 2, 1, per_tile // 2)


def _run_scatter(src, m1, m2, m3, m4, tails, Nn):
    E = src.shape[0]
    assert E % _SCAT_K == 0
    Np = ((Nn + 127) // 128) * 128
    rows_per_tile = Np // 16
    z128 = jnp.zeros((rows_per_tile, 128), jnp.float32)
    mesh = plsc.VectorSubcoreMesh(core_axis_name="c", subcore_axis_name="s")
    f = pl.kernel(
        _scatter_body,
        out_type=[jax.ShapeDtypeStruct((Np, 128), jnp.float32)] * 6,
        mesh=mesh,
        scratch_types=[
            pltpu.VMEM_SHARED((Np, 128), jnp.float32),
            pltpu.VMEM((_SCAT_K,), jnp.int32),
            pltpu.VMEM((_SCAT_K, 128), jnp.float32),
        ],
    )
    return f(src, m1, m2, m3, m4, tails, z128)


def kernel(x_h, x_g, edge_index, edge_attr, u, batch_h,
           W1a, b1a, W1b, b1b, W2a, b2a, W2b, b2b):
    src = edge_index[0]
    tgt = edge_index[1]
    Nn = x_h.shape[0]

    G = x_g[tgt]
    m1, m2, m3, m4, tails = _run_mlp1(G, edge_attr, W1a, b1a, W1b, b1b)

    S1, S2, S3, S4, T0, T1 = _run_scatter(src, m1, m2, m3, m4, tails, Nn)

    return _run_finalize(x_h, S1, S2, S3, S4, T0, T1, batch_h, u,
                         W2a, b2a, W2b, b2b, W1a.shape[0])


# speedup vs baseline: 4.8105x; 1.4140x over previous
import functools

import jax
import jax.numpy as jnp
from jax import lax
from jax.experimental import pallas as pl
from jax.experimental.pallas import tpu as pltpu
from jax.experimental.pallas import tpu_sc as plsc

N_H = 128
N_G = 128
N_X = 16
N_U = 16
NB_EDGE = 2048
NB_NODE = 512


def _leaky(x):
    return jnp.where(x >= 0, x, 0.1 * x)


_GATH_K = 512


def _gather_body(tgt_ref, xg_ref, g_ref, idx_v, rows_v, sem):
    E = tgt_ref.shape[0]
    n_chunks = E // _GATH_K
    cid = lax.axis_index("c")
    sid = lax.axis_index("s")
    wid = sid * 2 + cid
    n_slots = (n_chunks + 31) // 32

    def chunk(j, carry):
        g = j * 32 + wid

        @pl.when(g < n_chunks)
        def _():
            esl = pl.ds(g * _GATH_K, _GATH_K)
            pltpu.sync_copy(tgt_ref.at[esl], idx_v)
            pltpu.async_copy(xg_ref.at[idx_v], rows_v, sem).wait()
            pltpu.sync_copy(rows_v, g_ref.at[esl, :])

        return carry

    lax.fori_loop(0, n_slots, chunk, 0)


def _run_gather(x_g, tgt):
    E = tgt.shape[0]
    assert E % _GATH_K == 0
    mesh = plsc.VectorSubcoreMesh(core_axis_name="c", subcore_axis_name="s")
    f = pl.kernel(
        _gather_body,
        out_type=jax.ShapeDtypeStruct((E, N_G), jnp.float32),
        mesh=mesh,
        scratch_types=[
            pltpu.VMEM((_GATH_K,), jnp.int32),
            pltpu.VMEM((_GATH_K, N_G), jnp.float32),
            pltpu.SemaphoreType.DMA,
        ],
    )
    return f(tgt, x_g)


def _mlp1_body(g_ref, e_ref, w1a_ref, b1a_ref, w1b_ref, b1b_ref,
               m1_ref, m2_ref, m3_ref, m4_ref, tl_ref):
    x = jnp.concatenate([g_ref[...], e_ref[...]], axis=1)
    h = _leaky(jnp.dot(x, w1a_ref[...], preferred_element_type=jnp.float32)
               + b1a_ref[...])
    out = (jnp.dot(h, w1b_ref[...], preferred_element_type=jnp.float32)
           + b1b_ref[...])
    nb = out.shape[0]
    o2 = out * out
    o3 = o2 * out
    o4 = o2 * o2
    m1_ref[...] = out[:, :128]
    m2_ref[...] = o2[:, :128]
    m3_ref[...] = o3[:, :128]
    m4_ref[...] = o4[:, :128]
    tl_ref[...] = jnp.concatenate(
        [out[:, 128:], o2[:, 128:], o3[:, 128:], o4[:, 128:],
         jnp.ones((nb, 1), jnp.float32),
         jnp.zeros((nb, 63), jnp.float32)], axis=1)


def _run_mlp1(G, edge_attr, W1a, b1a, W1b, b1b):
    E = G.shape[0]
    D1 = W1a.shape[0]
    grid = (pl.cdiv(E, NB_EDGE),)
    out_sds = jax.ShapeDtypeStruct((E, 128), jnp.float32)
    blk_e = pl.BlockSpec((NB_EDGE, 128), lambda i: (i, 0))
    full = lambda s: pl.BlockSpec(s, lambda i: tuple(0 for _ in s))
    return pl.pallas_call(
        _mlp1_body,
        grid=grid,
        in_specs=[
            pl.BlockSpec((NB_EDGE, N_G), lambda i: (i, 0)),
            pl.BlockSpec((NB_EDGE, N_X), lambda i: (i, 0)),
            full(W1a.shape),
            full((1, D1)),
            full(W1b.shape),
            full((1, D1)),
        ],
        out_specs=[blk_e] * 5,
        out_shape=[out_sds] * 5,
    )(G, edge_attr, W1a, b1a.reshape(1, D1), W1b, b1b.reshape(1, D1))


def _fin_body(xh_ref, s1_ref, s2_ref, s3_ref, s4_ref, t0_ref, t1_ref,
              bh_ref,
              u_ref, w2h_ref, w2n_ref, w2a_ref, w2b_ref, w2c_ref, w2d_ref,
              w2u_ref, b2a_ref, w2bb_ref, b2b_ref, out_ref):
    t = t0_ref[...] + t1_ref[...]
    n = t[:, 64:65]
    inv = 1.0 / jnp.maximum(n, 1.0)
    cat = lambda s, lo: jnp.concatenate([s, t[:, lo:lo + 16]], axis=1)
    a = cat(s1_ref[...], 0) * inv
    m2 = cat(s2_ref[...], 16) * inv
    m3 = cat(s3_ref[...], 32) * inv
    m4 = cat(s4_ref[...], 48) * inv
    a2 = a * a
    v = 1e-6 + jnp.maximum(m2 - a2, 0.0)
    b = jnp.sqrt(v)
    c = (m3 - 3.0 * a * m2 + 2.0 * a * a2) / (v * b)
    d = (m4 - 4.0 * a * m3 + 6.0 * a2 * m2 - 3.0 * a2 * a2) / (v * v)
    bh = bh_ref[...]
    oh = (bh == lax.broadcasted_iota(jnp.int32, (bh.shape[0], 8), 1)
          ).astype(jnp.float32)
    ub = jnp.dot(oh, u_ref[...], preferred_element_type=jnp.float32)
    dot = lambda x, w: jnp.dot(x, w, preferred_element_type=jnp.float32)
    h = (dot(xh_ref[...], w2h_ref[...]) + n * w2n_ref[...]
         + dot(a, w2a_ref[...]) + dot(b, w2b_ref[...])
         + dot(c, w2c_ref[...]) + dot(d, w2d_ref[...])
         + dot(ub, w2u_ref[...]) + b2a_ref[...])
    h = _leaky(h)
    out_ref[...] = dot(h, w2bb_ref[...]) + b2b_ref[...]


def _run_finalize(x_h, S1, S2, S3, S4, T0, T1, batch_h, u,
                  W2a, b2a, W2b, b2b, D1):
    N = x_h.shape[0]
    grid = (pl.cdiv(N, NB_NODE),)
    blk = lambda w: pl.BlockSpec((NB_NODE, w), lambda i: (i, 0))
    full = lambda s: pl.BlockSpec(s, lambda i: tuple(0 for _ in s))
    o = 0
    W2h = W2a[o:o + N_H]; o += N_H
    W2n = W2a[o:o + 1]; o += 1
    W2aa = W2a[o:o + D1]; o += D1
    W2ab = W2a[o:o + D1]; o += D1
    W2ac = W2a[o:o + D1]; o += D1
    W2ad = W2a[o:o + D1]; o += D1
    W2u = W2a[o:o + N_U]
    return pl.pallas_call(
        _fin_body,
        grid=grid,
        in_specs=[
            blk(N_H), blk(128), blk(128), blk(128), blk(128), blk(128),
            blk(128),
            pl.BlockSpec((NB_NODE, 1), lambda i: (i, 0)),
            full(u.shape), full(W2h.shape), full(W2n.shape),
            full(W2aa.shape), full(W2ab.shape), full(W2ac.shape),
            full(W2ad.shape), full(W2u.shape), full((1, N_H)),
            full(W2b.shape), full((1, N_H)),
        ],
        out_specs=blk(N_H),
        out_shape=jax.ShapeDtypeStruct((N, N_H), jnp.float32),
    )(x_h, S1, S2, S3, S4, T0, T1, batch_h.reshape(N, 1), u,
      W2h, W2n, W2aa, W2ab, W2ac, W2ad, W2u, b2a.reshape(1, N_H),
      W2b, b2b.reshape(1, N_H))


_SCAT_K = 256


def _scatter_body(src_ref, m1_ref, m2_ref, m3_ref, m4_ref, tl_ref,
                  z128_ref,
                  s1_ref, s2_ref, s3_ref, s4_ref, t0_ref, t1_ref,
                  acc, idx_v, rows_v):
    E = m1_ref.shape[0]
    Np = s1_ref.shape[0]
    n_tiles = 16
    n_chunks = E // _SCAT_K
    per_tile = (n_chunks + n_tiles - 1) // n_tiles
    rows_per_tile = Np // n_tiles
    cid = lax.axis_index("c")
    sid = lax.axis_index("s")
    node_sl = pl.ds(sid * rows_per_tile, rows_per_tile)

    def do_pass(m_ref, out_ref, stride, par, nch):
        pltpu.sync_copy(z128_ref, acc.at[node_sl, :])
        plsc.subcore_barrier()

        def chunk(i, carry):
            j = i * stride + par
            g = j * n_tiles + sid

            @pl.when(g < n_chunks)
            def _():
                esl = pl.ds(g * _SCAT_K, _SCAT_K)
                pltpu.sync_copy(src_ref.at[esl], idx_v)
                pltpu.sync_copy(m_ref.at[esl, :], rows_v)
                pltpu.sync_copy(rows_v, acc.at[idx_v], add=True)

            return carry

        lax.fori_loop(0, nch, chunk, 0)
        plsc.subcore_barrier()
        pltpu.sync_copy(acc.at[node_sl, :], out_ref.at[node_sl, :])

    @pl.when(cid == 0)
    def _():
        do_pass(m1_ref, s1_ref, 1, 0, per_tile)
        do_pass(m2_ref, s2_ref, 1, 0, per_tile)
        do_pass(tl_ref, t0_ref, 2, 0, (per_tile + 1) // 2)

    @pl.when(cid == 1)
    def _():
        do_pass(m3_ref, s3_ref, 1, 0, per_tile)
        do_pass(m4_ref, s4_ref, 1, 0, per_tile)
        do_pass(tl_ref, t1_ref, 2, 1, per_tile // 2)


def _run_scatter(src, m1, m2, m3, m4, tails, Nn):
    E = src.shape[0]
    assert E % _SCAT_K == 0
    Np = ((Nn + 127) // 128) * 128
    rows_per_tile = Np // 16
    z128 = jnp.zeros((rows_per_tile, 128), jnp.float32)
    mesh = plsc.VectorSubcoreMesh(core_axis_name="c", subcore_axis_name="s")
    f = pl.kernel(
        _scatter_body,
        out_type=[jax.ShapeDtypeStruct((Np, 128), jnp.float32)] * 6,
        mesh=mesh,
        scratch_types=[
            pltpu.VMEM_SHARED((Np, 128), jnp.float32),
            pltpu.VMEM((_SCAT_K,), jnp.int32),
            pltpu.VMEM((_SCAT_K, 128), jnp.float32),
        ],
    )
    return f(src, m1, m2, m3, m4, tails, z128)


def kernel(x_h, x_g, edge_index, edge_attr, u, batch_h,
           W1a, b1a, W1b, b1b, W2a, b2a, W2b, b2b):
    src = edge_index[0]
    tgt = edge_index[1]
    Nn = x_h.shape[0]

    G = _run_gather(x_g, tgt)
    m1, m2, m3, m4, tails = _run_mlp1(G, edge_attr, W1a, b1a, W1b, b1b)

    S1, S2, S3, S4, T0, T1 = _run_scatter(src, m1, m2, m3, m4, tails, Nn)

    return _run_finalize(x_h, S1, S2, S3, S4, T0, T1, batch_h, u,
                         W2a, b2a, W2b, b2b, W1a.shape[0])


# speedup vs baseline: 6.2194x; 1.2929x over previous
import functools

import jax
import jax.numpy as jnp
from jax import lax
from jax.experimental import pallas as pl
from jax.experimental.pallas import tpu as pltpu
from jax.experimental.pallas import tpu_sc as plsc

N_H = 128
N_G = 128
N_X = 16
N_U = 16
NB_EDGE = 2048
NB_NODE = 512


def _leaky(x):
    return jnp.where(x >= 0, x, 0.1 * x)


_GATH_K = 512


def _gather_body(tgt_ref, xg_ref, g_ref, idx_v, rows_v, sem):
    E = tgt_ref.shape[0]
    n_chunks = E // _GATH_K
    cid = lax.axis_index("c")
    sid = lax.axis_index("s")
    wid = sid * 2 + cid
    n_slots = (n_chunks + 31) // 32

    def chunk(j, carry):
        g = j * 32 + wid

        @pl.when(g < n_chunks)
        def _():
            esl = pl.ds(g * _GATH_K, _GATH_K)
            pltpu.sync_copy(tgt_ref.at[esl], idx_v)
            pltpu.async_copy(xg_ref.at[idx_v], rows_v, sem).wait()
            pltpu.sync_copy(rows_v, g_ref.at[esl, :])

        return carry

    lax.fori_loop(0, n_slots, chunk, 0)


def _run_gather(x_g, tgt):
    E = tgt.shape[0]
    assert E % _GATH_K == 0
    mesh = plsc.VectorSubcoreMesh(core_axis_name="c", subcore_axis_name="s")
    f = pl.kernel(
        _gather_body,
        out_type=jax.ShapeDtypeStruct((E, N_G), jnp.float32),
        mesh=mesh,
        scratch_types=[
            pltpu.VMEM((_GATH_K,), jnp.int32),
            pltpu.VMEM((_GATH_K, N_G), jnp.float32),
            pltpu.SemaphoreType.DMA,
        ],
    )
    return f(tgt, x_g)


def _mlp1_body(g_ref, e_ref, w1a_ref, b1a_ref, w1b_ref, b1b_ref,
               m1_ref, m2_ref, m3_ref, m4_ref, tl_ref):
    x = jnp.concatenate([g_ref[...], e_ref[...]], axis=1)
    h = _leaky(jnp.dot(x, w1a_ref[...], preferred_element_type=jnp.float32)
               + b1a_ref[...])
    out = (jnp.dot(h, w1b_ref[...], preferred_element_type=jnp.float32)
           + b1b_ref[...])
    nb = out.shape[0]
    o2 = out * out
    o3 = o2 * out
    o4 = o2 * o2
    m1_ref[...] = out[:, :128]
    m2_ref[...] = o2[:, :128]
    m3_ref[...] = o3[:, :128]
    m4_ref[...] = o4[:, :128]
    tl_ref[...] = jnp.concatenate(
        [out[:, 128:], o2[:, 128:], o3[:, 128:], o4[:, 128:],
         jnp.ones((nb, 1), jnp.float32),
         jnp.zeros((nb, 63), jnp.float32)], axis=1)


def _run_mlp1(G, edge_attr, W1a, b1a, W1b, b1b):
    E = G.shape[0]
    D1 = W1a.shape[0]
    grid = (pl.cdiv(E, NB_EDGE),)
    out_sds = jax.ShapeDtypeStruct((E, 128), jnp.float32)
    blk_e = pl.BlockSpec((NB_EDGE, 128), lambda i: (i, 0))
    full = lambda s: pl.BlockSpec(s, lambda i: tuple(0 for _ in s))
    return pl.pallas_call(
        _mlp1_body,
        grid=grid,
        in_specs=[
            pl.BlockSpec((NB_EDGE, N_G), lambda i: (i, 0)),
            pl.BlockSpec((NB_EDGE, N_X), lambda i: (i, 0)),
            full(W1a.shape),
            full((1, D1)),
            full(W1b.shape),
            full((1, D1)),
        ],
        out_specs=[blk_e] * 5,
        out_shape=[out_sds] * 5,
    )(G, edge_attr, W1a, b1a.reshape(1, D1), W1b, b1b.reshape(1, D1))


def _fin_body(xh_ref, s1_ref, s2_ref, s3_ref, s4_ref, t0_ref, t1_ref,
              bh_ref,
              u_ref, w2h_ref, w2n_ref, w2a_ref, w2b_ref, w2c_ref, w2d_ref,
              w2u_ref, b2a_ref, w2bb_ref, b2b_ref, out_ref):
    t = t0_ref[...] + t1_ref[...]
    n = t[:, 64:65]
    inv = 1.0 / jnp.maximum(n, 1.0)
    cat = lambda s, lo: jnp.concatenate([s, t[:, lo:lo + 16]], axis=1)
    a = cat(s1_ref[...], 0) * inv
    m2 = cat(s2_ref[...], 16) * inv
    m3 = cat(s3_ref[...], 32) * inv
    m4 = cat(s4_ref[...], 48) * inv
    a2 = a * a
    v = 1e-6 + jnp.maximum(m2 - a2, 0.0)
    b = jnp.sqrt(v)
    c = (m3 - 3.0 * a * m2 + 2.0 * a * a2) / (v * b)
    d = (m4 - 4.0 * a * m3 + 6.0 * a2 * m2 - 3.0 * a2 * a2) / (v * v)
    bh = bh_ref[...]
    oh = (bh == lax.broadcasted_iota(jnp.int32, (bh.shape[0], 8), 1)
          ).astype(jnp.float32)
    ub = jnp.dot(oh, u_ref[...], preferred_element_type=jnp.float32)
    dot = lambda x, w: jnp.dot(x, w, preferred_element_type=jnp.float32)
    h = (dot(xh_ref[...], w2h_ref[...]) + n * w2n_ref[...]
         + dot(a, w2a_ref[...]) + dot(b, w2b_ref[...])
         + dot(c, w2c_ref[...]) + dot(d, w2d_ref[...])
         + dot(ub, w2u_ref[...]) + b2a_ref[...])
    h = _leaky(h)
    out_ref[...] = dot(h, w2bb_ref[...]) + b2b_ref[...]


def _run_finalize(x_h, S1, S2, S3, S4, T0, T1, batch_h, u,
                  W2a, b2a, W2b, b2b, D1):
    N = x_h.shape[0]
    grid = (pl.cdiv(N, NB_NODE),)
    blk = lambda w: pl.BlockSpec((NB_NODE, w), lambda i: (i, 0))
    full = lambda s: pl.BlockSpec(s, lambda i: tuple(0 for _ in s))
    o = 0
    W2h = W2a[o:o + N_H]; o += N_H
    W2n = W2a[o:o + 1]; o += 1
    W2aa = W2a[o:o + D1]; o += D1
    W2ab = W2a[o:o + D1]; o += D1
    W2ac = W2a[o:o + D1]; o += D1
    W2ad = W2a[o:o + D1]; o += D1
    W2u = W2a[o:o + N_U]
    return pl.pallas_call(
        _fin_body,
        grid=grid,
        in_specs=[
            blk(N_H), blk(128), blk(128), blk(128), blk(128), blk(128),
            blk(128),
            pl.BlockSpec((NB_NODE, 1), lambda i: (i, 0)),
            full(u.shape), full(W2h.shape), full(W2n.shape),
            full(W2aa.shape), full(W2ab.shape), full(W2ac.shape),
            full(W2ad.shape), full(W2u.shape), full((1, N_H)),
            full(W2b.shape), full((1, N_H)),
        ],
        out_specs=blk(N_H),
        out_shape=jax.ShapeDtypeStruct((N, N_H), jnp.float32),
    )(x_h, S1, S2, S3, S4, T0, T1, batch_h.reshape(N, 1), u,
      W2h, W2n, W2aa, W2ab, W2ac, W2ad, W2u, b2a.reshape(1, N_H),
      W2b, b2b.reshape(1, N_H))


_SCAT_K = 160


def _scatter_body(src_ref, m1_ref, m2_ref, m3_ref, m4_ref, tl_ref,
                  z128_ref,
                  s1_ref, s2_ref, s3_ref, s4_ref, t0_ref, t1_ref,
                  acc, idx0, idx1, rows0, rows1, sem0, sem1):
    E = m1_ref.shape[0]
    Np = s1_ref.shape[0]
    n_tiles = 16
    n_chunks = E // _SCAT_K
    per_tile = n_chunks // n_tiles
    rows_per_tile = Np // n_tiles
    cid = lax.axis_index("c")
    sid = lax.axis_index("s")
    node_sl = pl.ds(sid * rows_per_tile, rows_per_tile)
    bufs = ((idx0, rows0, sem0), (idx1, rows1, sem1))

    def do_pass(m_ref, out_ref, stride, par, nch):
        pltpu.sync_copy(z128_ref, acc.at[node_sl, :])
        plsc.subcore_barrier()

        def issue(i, b):
            g = (i * stride + par) * n_tiles + sid
            esl = pl.ds(g * _SCAT_K, _SCAT_K)
            pltpu.async_copy(src_ref.at[esl], bufs[b][0], bufs[b][2])
            pltpu.async_copy(m_ref.at[esl, :], bufs[b][1], bufs[b][2])

        def drain(b):
            idx_v, rows_v, sem = bufs[b]
            pltpu.make_async_copy(src_ref.at[pl.ds(0, _SCAT_K)], idx_v,
                                  sem).wait()
            pltpu.make_async_copy(m_ref.at[pl.ds(0, _SCAT_K), :], rows_v,
                                  sem).wait()
            pltpu.sync_copy(rows_v, acc.at[idx_v], add=True)

        issue(0, 0)

        def pair(i2, carry):
            for b in (0, 1):
                i = 2 * i2 + b

                @pl.when(i < nch)
                def _():
                    @pl.when(i + 1 < nch)
                    def _():
                        issue(i + 1, 1 - b)

                    drain(b)

            return carry

        lax.fori_loop(0, (nch + 1) // 2, pair, 0)
        plsc.subcore_barrier()
        pltpu.sync_copy(acc.at[node_sl, :], out_ref.at[node_sl, :])

    @pl.when(cid == 0)
    def _():
        do_pass(m1_ref, s1_ref, 1, 0, per_tile)
        do_pass(m2_ref, s2_ref, 1, 0, per_tile)
        do_pass(tl_ref, t0_ref, 2, 0, (per_tile + 1) // 2)

    @pl.when(cid == 1)
    def _():
        do_pass(m3_ref, s3_ref, 1, 0, per_tile)
        do_pass(m4_ref, s4_ref, 1, 0, per_tile)
        do_pass(tl_ref, t1_ref, 2, 1, per_tile // 2)


def _run_scatter(src, m1, m2, m3, m4, tails, Nn):
    E = src.shape[0]
    assert E % _SCAT_K == 0
    Np = ((Nn + 127) // 128) * 128
    rows_per_tile = Np // 16
    z128 = jnp.zeros((rows_per_tile, 128), jnp.float32)
    mesh = plsc.VectorSubcoreMesh(core_axis_name="c", subcore_axis_name="s")
    f = pl.kernel(
        _scatter_body,
        out_type=[jax.ShapeDtypeStruct((Np, 128), jnp.float32)] * 6,
        mesh=mesh,
        scratch_types=[
            pltpu.VMEM_SHARED((Np, 128), jnp.float32),
            pltpu.VMEM((_SCAT_K,), jnp.int32),
            pltpu.VMEM((_SCAT_K,), jnp.int32),
            pltpu.VMEM((_SCAT_K, 128), jnp.float32),
            pltpu.VMEM((_SCAT_K, 128), jnp.float32),
            pltpu.SemaphoreType.DMA,
            pltpu.SemaphoreType.DMA,
        ],
    )
    return f(src, m1, m2, m3, m4, tails, z128)


def kernel(x_h, x_g, edge_index, edge_attr, u, batch_h,
           W1a, b1a, W1b, b1b, W2a, b2a, W2b, b2b):
    src = edge_index[0]
    tgt = edge_index[1]
    Nn = x_h.shape[0]

    G = _run_gather(x_g, tgt)
    m1, m2, m3, m4, tails = _run_mlp1(G, edge_attr, W1a, b1a, W1b, b1b)

    S1, S2, S3, S4, T0, T1 = _run_scatter(src, m1, m2, m3, m4, tails, Nn)

    return _run_finalize(x_h, S1, S2, S3, S4, T0, T1, batch_h, u,
                         W2a, b2a, W2b, b2b, W1a.shape[0])


# speedup vs baseline: 6.2546x; 1.0057x over previous
import functools

import jax
import jax.numpy as jnp
from jax import lax
from jax.experimental import pallas as pl
from jax.experimental.pallas import tpu as pltpu
from jax.experimental.pallas import tpu_sc as plsc

N_H = 128
N_G = 128
N_X = 16
N_U = 16
NB_EDGE = 2048
NB_NODE = 512


def _leaky(x):
    return jnp.where(x >= 0, x, 0.1 * x)


_GATH_K = 400


def _gather_body(tgt_ref, xg_ref, g_ref,
                 idx0, idx1, rows0, rows1, gsem0, gsem1, osem0, osem1):
    E = tgt_ref.shape[0]
    cid = lax.axis_index("c")
    sid = lax.axis_index("s")
    wid = sid * 2 + cid
    n_slots = E // _GATH_K // 32
    bufs = ((idx0, rows0, gsem0, osem0), (idx1, rows1, gsem1, osem1))

    def slot(j):
        return pl.ds((j * 32 + wid) * _GATH_K, _GATH_K)

    def load_and_gather(j, b):
        idx_v, rows_v, gsem, _ = bufs[b]
        pltpu.sync_copy(tgt_ref.at[slot(j)], idx_v)
        pltpu.async_copy(xg_ref.at[idx_v], rows_v, gsem)

    def wait_gather(b):
        idx_v, rows_v, gsem, _ = bufs[b]
        pltpu.make_async_copy(xg_ref.at[idx_v], rows_v, gsem).wait()

    def writeback(j, b):
        rows_v, osem = bufs[b][1], bufs[b][3]
        pltpu.async_copy(rows_v, g_ref.at[slot(j)], osem)

    def wait_writeback(b):
        rows_v, osem = bufs[b][1], bufs[b][3]
        pltpu.make_async_copy(rows_v, g_ref.at[pl.ds(0, _GATH_K)],
                              osem).wait()

    load_and_gather(0, 0)

    def pair(j2, carry):
        for b in (0, 1):
            j = 2 * j2 + b

            @pl.when(j < n_slots)
            def _():
                wait_gather(b)
                writeback(j, b)

                @pl.when(j + 1 < n_slots)
                def _():
                    @pl.when(j >= 1)
                    def _():
                        wait_writeback(1 - b)

                    load_and_gather(j + 1, 1 - b)

        return carry

    lax.fori_loop(0, (n_slots + 1) // 2, pair, 0)
    wait_writeback((n_slots - 1) % 2)
    if n_slots > 1:
        wait_writeback((n_slots - 2) % 2)


def _run_gather(x_g, tgt):
    E = tgt.shape[0]
    assert E % (_GATH_K * 32) == 0
    mesh = plsc.VectorSubcoreMesh(core_axis_name="c", subcore_axis_name="s")
    f = pl.kernel(
        _gather_body,
        out_type=jax.ShapeDtypeStruct((E, N_G), jnp.float32),
        mesh=mesh,
        scratch_types=[
            pltpu.VMEM((_GATH_K,), jnp.int32),
            pltpu.VMEM((_GATH_K,), jnp.int32),
            pltpu.VMEM((_GATH_K, N_G), jnp.float32),
            pltpu.VMEM((_GATH_K, N_G), jnp.float32),
            pltpu.SemaphoreType.DMA,
            pltpu.SemaphoreType.DMA,
            pltpu.SemaphoreType.DMA,
            pltpu.SemaphoreType.DMA,
        ],
    )
    return f(tgt, x_g)


def _mlp1_body(g_ref, e_ref, w1a_ref, b1a_ref, w1b_ref, b1b_ref,
               m1_ref, m2_ref, m3_ref, m4_ref, tl_ref):
    x = jnp.concatenate([g_ref[...], e_ref[...]], axis=1)
    h = _leaky(jnp.dot(x, w1a_ref[...], preferred_element_type=jnp.float32)
               + b1a_ref[...])
    out = (jnp.dot(h, w1b_ref[...], preferred_element_type=jnp.float32)
           + b1b_ref[...])
    nb = out.shape[0]
    o2 = out * out
    o3 = o2 * out
    o4 = o2 * o2
    m1_ref[...] = out[:, :128]
    m2_ref[...] = o2[:, :128]
    m3_ref[...] = o3[:, :128]
    m4_ref[...] = o4[:, :128]
    tl_ref[...] = jnp.concatenate(
        [out[:, 128:], o2[:, 128:], o3[:, 128:], o4[:, 128:],
         jnp.ones((nb, 1), jnp.float32),
         jnp.zeros((nb, 63), jnp.float32)], axis=1)


def _run_mlp1(G, edge_attr, W1a, b1a, W1b, b1b):
    E = G.shape[0]
    D1 = W1a.shape[0]
    grid = (pl.cdiv(E, NB_EDGE),)
    out_sds = jax.ShapeDtypeStruct((E, 128), jnp.float32)
    blk_e = pl.BlockSpec((NB_EDGE, 128), lambda i: (i, 0))
    full = lambda s: pl.BlockSpec(s, lambda i: tuple(0 for _ in s))
    return pl.pallas_call(
        _mlp1_body,
        grid=grid,
        in_specs=[
            pl.BlockSpec((NB_EDGE, N_G), lambda i: (i, 0)),
            pl.BlockSpec((NB_EDGE, N_X), lambda i: (i, 0)),
            full(W1a.shape),
            full((1, D1)),
            full(W1b.shape),
            full((1, D1)),
        ],
        out_specs=[blk_e] * 5,
        out_shape=[out_sds] * 5,
    )(G, edge_attr, W1a, b1a.reshape(1, D1), W1b, b1b.reshape(1, D1))


def _fin_body(xh_ref, s1_ref, s2_ref, s3_ref, s4_ref, t0_ref, t1_ref,
              bh_ref,
              u_ref, w2h_ref, w2n_ref, w2a_ref, w2b_ref, w2c_ref, w2d_ref,
              w2u_ref, b2a_ref, w2bb_ref, b2b_ref, out_ref):
    t = t0_ref[...] + t1_ref[...]
    n = t[:, 64:65]
    inv = 1.0 / jnp.maximum(n, 1.0)
    cat = lambda s, lo: jnp.concatenate([s, t[:, lo:lo + 16]], axis=1)
    a = cat(s1_ref[...], 0) * inv
    m2 = cat(s2_ref[...], 16) * inv
    m3 = cat(s3_ref[...], 32) * inv
    m4 = cat(s4_ref[...], 48) * inv
    a2 = a * a
    v = 1e-6 + jnp.maximum(m2 - a2, 0.0)
    b = jnp.sqrt(v)
    c = (m3 - 3.0 * a * m2 + 2.0 * a * a2) / (v * b)
    d = (m4 - 4.0 * a * m3 + 6.0 * a2 * m2 - 3.0 * a2 * a2) / (v * v)
    bh = bh_ref[...]
    oh = (bh == lax.broadcasted_iota(jnp.int32, (bh.shape[0], 8), 1)
          ).astype(jnp.float32)
    ub = jnp.dot(oh, u_ref[...], preferred_element_type=jnp.float32)
    dot = lambda x, w: jnp.dot(x, w, preferred_element_type=jnp.float32)
    h = (dot(xh_ref[...], w2h_ref[...]) + n * w2n_ref[...]
         + dot(a, w2a_ref[...]) + dot(b, w2b_ref[...])
         + dot(c, w2c_ref[...]) + dot(d, w2d_ref[...])
         + dot(ub, w2u_ref[...]) + b2a_ref[...])
    h = _leaky(h)
    out_ref[...] = dot(h, w2bb_ref[...]) + b2b_ref[...]


def _run_finalize(x_h, S1, S2, S3, S4, T0, T1, batch_h, u,
                  W2a, b2a, W2b, b2b, D1):
    N = x_h.shape[0]
    grid = (pl.cdiv(N, NB_NODE),)
    blk = lambda w: pl.BlockSpec((NB_NODE, w), lambda i: (i, 0))
    full = lambda s: pl.BlockSpec(s, lambda i: tuple(0 for _ in s))
    o = 0
    W2h = W2a[o:o + N_H]; o += N_H
    W2n = W2a[o:o + 1]; o += 1
    W2aa = W2a[o:o + D1]; o += D1
    W2ab = W2a[o:o + D1]; o += D1
    W2ac = W2a[o:o + D1]; o += D1
    W2ad = W2a[o:o + D1]; o += D1
    W2u = W2a[o:o + N_U]
    return pl.pallas_call(
        _fin_body,
        grid=grid,
        in_specs=[
            blk(N_H), blk(128), blk(128), blk(128), blk(128), blk(128),
            blk(128),
            pl.BlockSpec((NB_NODE, 1), lambda i: (i, 0)),
            full(u.shape), full(W2h.shape), full(W2n.shape),
            full(W2aa.shape), full(W2ab.shape), full(W2ac.shape),
            full(W2ad.shape), full(W2u.shape), full((1, N_H)),
            full(W2b.shape), full((1, N_H)),
        ],
        out_specs=blk(N_H),
        out_shape=jax.ShapeDtypeStruct((N, N_H), jnp.float32),
    )(x_h, S1, S2, S3, S4, T0, T1, batch_h.reshape(N, 1), u,
      W2h, W2n, W2aa, W2ab, W2ac, W2ad, W2u, b2a.reshape(1, N_H),
      W2b, b2b.reshape(1, N_H))


_SCAT_K = 160


def _scatter_body(src_ref, m1_ref, m2_ref, m3_ref, m4_ref, tl_ref,
                  z128_ref,
                  s1_ref, s2_ref, s3_ref, s4_ref, t0_ref, t1_ref,
                  acc, idx0, idx1, rows0, rows1, sem0, sem1):
    E = m1_ref.shape[0]
    Np = s1_ref.shape[0]
    n_tiles = 16
    n_chunks = E // _SCAT_K
    per_tile = n_chunks // n_tiles
    rows_per_tile = Np // n_tiles
    cid = lax.axis_index("c")
    sid = lax.axis_index("s")
    node_sl = pl.ds(sid * rows_per_tile, rows_per_tile)
    bufs = ((idx0, rows0, sem0), (idx1, rows1, sem1))

    def do_pass(m_ref, out_ref, stride, par, nch):
        pltpu.sync_copy(z128_ref, acc.at[node_sl, :])
        plsc.subcore_barrier()

        def issue(i, b):
            g = (i * stride + par) * n_tiles + sid
            esl = pl.ds(g * _SCAT_K, _SCAT_K)
            pltpu.async_copy(src_ref.at[esl], bufs[b][0], bufs[b][2])
            pltpu.async_copy(m_ref.at[esl, :], bufs[b][1], bufs[b][2])

        def drain(b):
            idx_v, rows_v, sem = bufs[b]
            pltpu.make_async_copy(src_ref.at[pl.ds(0, _SCAT_K)], idx_v,
                                  sem).wait()
            pltpu.make_async_copy(m_ref.at[pl.ds(0, _SCAT_K), :], rows_v,
                                  sem).wait()
            pltpu.sync_copy(rows_v, acc.at[idx_v], add=True)

        issue(0, 0)

        def pair(i2, carry):
            for b in (0, 1):
                i = 2 * i2 + b

                @pl.when(i < nch)
                def _():
                    @pl.when(i + 1 < nch)
                    def _():
                        issue(i + 1, 1 - b)

                    drain(b)

            return carry

        lax.fori_loop(0, (nch + 1) // 2, pair, 0)
        plsc.subcore_barrier()
        pltpu.sync_copy(acc.at[node_sl, :], out_ref.at[node_sl, :])

    @pl.when(cid == 0)
    def _():
        do_pass(m1_ref, s1_ref, 1, 0, per_tile)
        do_pass(m2_ref, s2_ref, 1, 0, per_tile)
        do_pass(tl_ref, t0_ref, 2, 0, (per_tile + 1) // 2)

    @pl.when(cid == 1)
    def _():
        do_pass(m3_ref, s3_ref, 1, 0, per_tile)
        do_pass(m4_ref, s4_ref, 1, 0, per_tile)
        do_pass(tl_ref, t1_ref, 2, 1, per_tile // 2)


def _run_scatter(src, m1, m2, m3, m4, tails, Nn):
    E = src.shape[0]
    assert E % _SCAT_K == 0
    Np = ((Nn + 127) // 128) * 128
    rows_per_tile = Np // 16
    z128 = jnp.zeros((rows_per_tile, 128), jnp.float32)
    mesh = plsc.VectorSubcoreMesh(core_axis_name="c", subcore_axis_name="s")
    f = pl.kernel(
        _scatter_body,
        out_type=[jax.ShapeDtypeStruct((Np, 128), jnp.float32)] * 6,
        mesh=mesh,
        scratch_types=[
            pltpu.VMEM_SHARED((Np, 128), jnp.float32),
            pltpu.VMEM((_SCAT_K,), jnp.int32),
            pltpu.VMEM((_SCAT_K,), jnp.int32),
            pltpu.VMEM((_SCAT_K, 128), jnp.float32),
            pltpu.VMEM((_SCAT_K, 128), jnp.float32),
            pltpu.SemaphoreType.DMA,
            pltpu.SemaphoreType.DMA,
        ],
    )
    return f(src, m1, m2, m3, m4, tails, z128)


def kernel(x_h, x_g, edge_index, edge_attr, u, batch_h,
           W1a, b1a, W1b, b1b, W2a, b2a, W2b, b2b):
    src = edge_index[0]
    tgt = edge_index[1]
    Nn = x_h.shape[0]

    G = _run_gather(x_g, tgt)
    m1, m2, m3, m4, tails = _run_mlp1(G, edge_attr, W1a, b1a, W1b, b1b)

    S1, S2, S3, S4, T0, T1 = _run_scatter(src, m1, m2, m3, m4, tails, Nn)

    return _run_finalize(x_h, S1, S2, S3, S4, T0, T1, batch_h, u,
                         W2a, b2a, W2b, b2b, W1a.shape[0])
